# baseline (speedup 1.0000x reference)
import jax
import jax.numpy as jnp
from jax import lax
from jax.experimental import pallas as pl
from jax.experimental.pallas import tpu as pltpu
from jax.experimental.pallas import tpu_sc as plsc

_N = 10000
_E = 320000
_D = 128
_L = 3
_G = 256
_EPS = 1e-5

_NC = 2
_NS = 16
_NW = _NC * _NS
_EPW = _E // _NW
_K = 80
_B = 4
_SM = _EPW // _K
_NQ = _SM // _B
_NPAD = 10240
_RPT = _NPAD // _NS
_DW = 16

_F32 = jnp.float32
_HI = lax.Precision.HIGHEST

_sc_mesh = plsc.VectorSubcoreMesh(
    core_axis_name="c", subcore_axis_name="s", num_cores=_NC, num_subcores=_NS
)


def _sc_deg_body(dstm_hbm, out_hbm, idx_v, zero_v, one_v, acc, sem):
    c = lax.axis_index("c")
    s = lax.axis_index("s")
    wid = c * _NS + s

    @pl.loop(0, _K)
    def _fill(i):
        zero_v[i, :] = jnp.zeros((_DW,), _F32)
        one_v[i, :] = jnp.ones((_DW,), _F32)

    for t in range(_RPT // _K):
        pltpu.sync_copy(zero_v, acc.at[pl.ds(s * _RPT + t * _K, _K)])
    plsc.subcore_barrier()

    pltpu.sync_copy(dstm_hbm.at[wid], idx_v)

    @pl.loop(0, _NQ)
    def _scat(qq):
        j0 = _B * qq
        ss = [pltpu.async_copy(one_v, acc.at[idx_v.at[j0 + q]], sem, add=True)
              for q in range(_B)]
        for d in ss:
            d.wait()

    pltpu.sync_copy(one_v, acc.at[idx_v.at[_SM - 1]], add=True)

    plsc.subcore_barrier()
    pltpu.sync_copy(acc.at[pl.ds(s * _RPT, _RPT)], out_hbm.at[c, pl.ds(s * _RPT, _RPT)])


_deg_call = pl.kernel(
    _sc_deg_body,
    out_type=jax.ShapeDtypeStruct((_NC, _NPAD, _DW), _F32),
    mesh=_sc_mesh,
    scratch_types=[
        pltpu.VMEM((_SM, _K), jnp.int32),
        pltpu.VMEM((_K, _DW), _F32),
        pltpu.VMEM((_K, _DW), _F32),
        pltpu.VMEM_SHARED((_NPAD, _DW), _F32),
        pltpu.SemaphoreType.DMA,
    ],
)


def _sc_edge_body(p_hbm, srcm_hbm, srcl_hbm, dstm_hbm, dstl_hbm, out_hbm,
                  sbuf, dbuf, buf, acc, gs, ss):
    c = lax.axis_index("c")
    s = lax.axis_index("s")
    wid = c * _NS + s

    @pl.loop(0, _K)
    def _fill(i):
        for dd in range(_D // 16):
            buf[0, i, pl.ds(dd * 16, 16)] = jnp.zeros((16,), _F32)

    for t in range(_RPT // _K):
        pltpu.sync_copy(buf.at[0], acc.at[pl.ds(s * _RPT + t * _K, _K)])
    plsc.subcore_barrier()

    @pl.loop(0, _NQ)
    def _edge(qq):
        pltpu.sync_copy(srcm_hbm.at[wid, qq], sbuf)
        gg = [pltpu.async_copy(p_hbm.at[sbuf.at[q]], buf.at[q], gs)
              for q in range(_B)]
        pltpu.sync_copy(dstm_hbm.at[wid, qq], dbuf)
        for d in gg:
            d.wait()
        sc = [pltpu.async_copy(buf.at[q], acc.at[dbuf.at[q]], ss, add=True)
              for q in range(_B)]
        for d in sc:
            d.wait()

    pltpu.sync_copy(srcl_hbm.at[wid], sbuf.at[0])
    pltpu.sync_copy(dstl_hbm.at[wid], dbuf.at[0])
    pltpu.async_copy(p_hbm.at[sbuf.at[0]], buf.at[0], gs).wait()
    pltpu.sync_copy(buf.at[0], acc.at[dbuf.at[0]], add=True)

    plsc.subcore_barrier()
    pltpu.sync_copy(acc.at[pl.ds(s * _RPT, _RPT)], out_hbm.at[c, pl.ds(s * _RPT, _RPT)])


_edge_call = pl.kernel(
    _sc_edge_body,
    out_type=jax.ShapeDtypeStruct((_NC, _NPAD, _D), _F32),
    mesh=_sc_mesh,
    scratch_types=[
        pltpu.VMEM((_B, _K), jnp.int32),
        pltpu.VMEM((_B, _K), jnp.int32),
        pltpu.VMEM((_B, _K, _D), _F32),
        pltpu.VMEM_SHARED((_NPAD, _D), _F32),
        pltpu.SemaphoreType.DMA,
        pltpu.SemaphoreType.DMA,
    ],
)


def _dinv_from(deg2):
    deg = deg2[0, :_N] + deg2[1, :_N] + 1.0
    return (1.0 / jnp.sqrt(deg))[:, None]


def _tc_emb_body(x_ref, We_ref, be_ref, W0_ref, deg_ref, h_ref, p_ref):
    dinv = _dinv_from(deg_ref[...])
    h = jnp.dot(x_ref[...], We_ref[...], precision=_HI, preferred_element_type=_F32)
    h = h + be_ref[...][None, :]
    hw = jnp.dot(h, W0_ref[...], precision=_HI, preferred_element_type=_F32)
    h_ref[...] = h
    p_ref[...] = hw * dinv


_tc_emb = pl.pallas_call(
    _tc_emb_body,
    out_shape=(
        jax.ShapeDtypeStruct((_N, _D), _F32),
        jax.ShapeDtypeStruct((_N, _D), _F32),
    ),
    compiler_params=pltpu.CompilerParams(vmem_limit_bytes=100 * 1024 * 1024),
)


def _post_norm(sp_ref, p_ref, h_ref, deg_ref, b_ref, g_ref, bt_ref):
    dinv = _dinv_from(deg_ref[...])
    sp = sp_ref[...]
    agg = dinv * (sp[0, :_N, :] + sp[1, :_N, :] + p_ref[...]) + b_ref[...][None, :]
    mu = jnp.mean(agg, axis=0)
    xc = agg - mu[None, :]
    var = jnp.mean(xc * xc, axis=0)
    y = xc * lax.rsqrt(var + _EPS)[None, :] * g_ref[...][None, :] + bt_ref[...][None, :]
    return jnp.maximum(y, 0.0) + h_ref[...], dinv


def _tc_norm_body(sp_ref, p_ref, h_ref, deg_ref, b_ref, g_ref, bt_ref, Wn_ref,
                  hn_ref, pn_ref):
    hn, dinv = _post_norm(sp_ref, p_ref, h_ref, deg_ref, b_ref, g_ref, bt_ref)
    hn_ref[...] = hn
    hw = jnp.dot(hn, Wn_ref[...], precision=_HI, preferred_element_type=_F32)
    pn_ref[...] = hw * dinv


_tc_norm = pl.pallas_call(
    _tc_norm_body,
    out_shape=(
        jax.ShapeDtypeStruct((_N, _D), _F32),
        jax.ShapeDtypeStruct((_N, _D), _F32),
    ),
    compiler_params=pltpu.CompilerParams(vmem_limit_bytes=100 * 1024 * 1024),
)


def _tc_final_body(sp_ref, p_ref, h_ref, deg_ref, b_ref, g_ref, bt_ref,
                   batch_ref, W1_ref, b1_ref, W2_ref, b2_ref, out_ref):
    hn, _ = _post_norm(sp_ref, p_ref, h_ref, deg_ref, b_ref, g_ref, bt_ref)
    seg = lax.broadcasted_iota(jnp.int32, (_G, _N), 0)
    onehot = (seg == batch_ref[...]).astype(_F32)
    sums = jnp.dot(onehot, hn, precision=_HI, preferred_element_type=_F32)
    counts = jnp.sum(onehot, axis=1)
    pooled = sums / jnp.maximum(counts, 1.0)[:, None]
    o = jnp.dot(pooled, W1_ref[...], precision=_HI, preferred_element_type=_F32)
    o = jnp.maximum(o + b1_ref[...][None, :], 0.0)
    o = jnp.dot(o, W2_ref[...], precision=_HI, preferred_element_type=_F32)
    out_ref[...] = o + b2_ref[...][None, :]


_tc_final = pl.pallas_call(
    _tc_final_body,
    out_shape=jax.ShapeDtypeStruct((_G, 1), _F32),
    compiler_params=pltpu.CompilerParams(vmem_limit_bytes=100 * 1024 * 1024),
)


def kernel(x, edge_index, batch, W_emb, b_emb, W_convs, b_convs, gammas, betas,
           W1, b1, W2, b2):
    srcw = edge_index[0].reshape(_NW, _EPW)
    dstw = edge_index[1].reshape(_NW, _EPW)
    srcm4 = srcw[:, : _B * _NQ * _K].reshape(_NW, _NQ, _B, _K)
    srcl = srcw[:, _B * _NQ * _K :]
    dstm4 = dstw[:, : _B * _NQ * _K].reshape(_NW, _NQ, _B, _K)
    dstl = dstw[:, _B * _NQ * _K :]
    degp = _deg_call(dstw.reshape(_NW, _SM, _K))
    deg2 = degp[:, :, 0]
    h, p = _tc_emb(x, W_emb, b_emb, W_convs[0], deg2)
    out = None
    for i in range(_L):
        sp = _edge_call(p, srcm4, srcl, dstm4, dstl)
        if i < _L - 1:
            h, p = _tc_norm(sp, p, h, deg2, b_convs[i], gammas[i], betas[i],
                            W_convs[i + 1])
        else:
            out = _tc_final(sp, p, h, deg2, b_convs[i], gammas[i], betas[i],
                            batch.reshape(1, _N), W1, b1, W2, b2)
    return out


# speedup vs baseline: 1.0286x; 1.0286x over previous
import jax
import jax.numpy as jnp
from jax import lax
from jax.experimental import pallas as pl
from jax.experimental.pallas import tpu as pltpu
from jax.experimental.pallas import tpu_sc as plsc

_N = 10000
_E = 320000
_D = 128
_L = 3
_G = 256
_EPS = 1e-5

_NC = 2
_NS = 16
_NW = _NC * _NS
_EPW = _E // _NW

_KE = 128
_BE = 3
_QE = 26
_KT = _EPW - _QE * _BE * _KE
_R0 = 632
_R15 = _N - 15 * _R0

_KD = 80
_SD = _EPW // _KD
_BD = 4
_QD = _SD // _BD
_NPAD = 10240
_RPT = _NPAD // _NS
_DW = 16

_F32 = jnp.float32
_HI = lax.Precision.HIGHEST

_sc_mesh = plsc.VectorSubcoreMesh(
    core_axis_name="c", subcore_axis_name="s", num_cores=_NC, num_subcores=_NS
)


def _sc_deg_body(dstm_hbm, out_hbm, idx_v, zero_v, one_v, acc, sem):
    c = lax.axis_index("c")
    s = lax.axis_index("s")
    wid = c * _NS + s

    @pl.loop(0, _KD)
    def _fill(i):
        zero_v[i, :] = jnp.zeros((_DW,), _F32)
        one_v[i, :] = jnp.ones((_DW,), _F32)

    for t in range(_RPT // _KD):
        pltpu.sync_copy(zero_v, acc.at[pl.ds(s * _RPT + t * _KD, _KD)])
    plsc.subcore_barrier()

    pltpu.sync_copy(dstm_hbm.at[wid], idx_v)

    @pl.loop(0, _QD)
    def _scat(qq):
        j0 = _BD * qq
        ss = [pltpu.async_copy(one_v, acc.at[idx_v.at[j0 + q]], sem, add=True)
              for q in range(_BD)]
        for d in ss:
            d.wait()

    pltpu.sync_copy(one_v, acc.at[idx_v.at[_SD - 1]], add=True)

    plsc.subcore_barrier()
    pltpu.sync_copy(acc.at[pl.ds(s * _RPT, _RPT)], out_hbm.at[c, pl.ds(s * _RPT, _RPT)])


_deg_call = pl.kernel(
    _sc_deg_body,
    out_type=jax.ShapeDtypeStruct((_NC, _NPAD, _DW), _F32),
    mesh=_sc_mesh,
    scratch_types=[
        pltpu.VMEM((_SD, _KD), jnp.int32),
        pltpu.VMEM((_KD, _DW), _F32),
        pltpu.VMEM((_KD, _DW), _F32),
        pltpu.VMEM_SHARED((_NPAD, _DW), _F32),
        pltpu.SemaphoreType.DMA,
    ],
)


def _sc_edge_body(p_hbm, srcm_hbm, srct_hbm, dstm_hbm, dstt_hbm, out_hbm,
                  sbuf, dbuf, st_v, dt_v, buf, acc, gs, ss):
    c = lax.axis_index("c")
    s = lax.axis_index("s")
    wid = c * _NS + s

    @pl.loop(0, _KE)
    def _fill(i):
        for dd in range(_D // 16):
            buf[0, i, pl.ds(dd * 16, 16)] = jnp.zeros((16,), _F32)

    @pl.when(s < _NS - 1)
    def _z0():
        for t in range(4):
            pltpu.sync_copy(buf.at[0], acc.at[pl.ds(s * _R0 + t * _KE, _KE)])
        pltpu.sync_copy(buf.at[0, pl.ds(0, _R0 - 512)],
                        acc.at[pl.ds(s * _R0 + 512, _R0 - 512)])

    @pl.when(s == _NS - 1)
    def _z15():
        for t in range(4):
            pltpu.sync_copy(buf.at[0], acc.at[pl.ds(15 * _R0 + t * _KE, _KE)])
        pltpu.sync_copy(buf.at[0, pl.ds(0, _R15 - 512)],
                        acc.at[pl.ds(15 * _R0 + 512, _R15 - 512)])

    plsc.subcore_barrier()

    @pl.loop(0, _QE)
    def _edge(qq):
        pltpu.sync_copy(srcm_hbm.at[wid, qq], sbuf)
        gg = [pltpu.async_copy(p_hbm.at[sbuf.at[q]], buf.at[q], gs)
              for q in range(_BE)]
        pltpu.sync_copy(dstm_hbm.at[wid, qq], dbuf)
        for d in gg:
            d.wait()
        sc = [pltpu.async_copy(buf.at[q], acc.at[dbuf.at[q]], ss, add=True)
              for q in range(_BE)]
        for d in sc:
            d.wait()

    pltpu.sync_copy(srct_hbm.at[wid], st_v)
    pltpu.sync_copy(dstt_hbm.at[wid], dt_v)
    pltpu.async_copy(p_hbm.at[st_v], buf.at[0, pl.ds(0, _KT)], gs).wait()
    pltpu.sync_copy(buf.at[0, pl.ds(0, _KT)], acc.at[dt_v], add=True)

    plsc.subcore_barrier()

    @pl.when(s < _NS - 1)
    def _r0():
        pltpu.sync_copy(acc.at[pl.ds(s * _R0, _R0)],
                        out_hbm.at[c, pl.ds(s * _R0, _R0)])

    @pl.when(s == _NS - 1)
    def _r15():
        pltpu.sync_copy(acc.at[pl.ds(15 * _R0, _R15)],
                        out_hbm.at[c, pl.ds(15 * _R0, _R15)])


_edge_call = pl.kernel(
    _sc_edge_body,
    out_type=jax.ShapeDtypeStruct((_NC, _N, _D), _F32),
    mesh=_sc_mesh,
    scratch_types=[
        pltpu.VMEM((_BE, _KE), jnp.int32),
        pltpu.VMEM((_BE, _KE), jnp.int32),
        pltpu.VMEM((_KT,), jnp.int32),
        pltpu.VMEM((_KT,), jnp.int32),
        pltpu.VMEM((_BE, _KE, _D), _F32),
        pltpu.VMEM_SHARED((_N, _D), _F32),
        pltpu.SemaphoreType.DMA,
        pltpu.SemaphoreType.DMA,
    ],
)


def _dinv_from(deg2):
    deg = deg2[0, :_N] + deg2[1, :_N] + 1.0
    return (1.0 / jnp.sqrt(deg))[:, None]


def _tc_emb_body(x_ref, We_ref, be_ref, W0_ref, deg_ref, h_ref, p_ref):
    dinv = _dinv_from(deg_ref[...])
    h = jnp.dot(x_ref[...], We_ref[...], precision=_HI, preferred_element_type=_F32)
    h = h + be_ref[...][None, :]
    hw = jnp.dot(h, W0_ref[...], precision=_HI, preferred_element_type=_F32)
    h_ref[...] = h
    p_ref[...] = hw * dinv


_tc_emb = pl.pallas_call(
    _tc_emb_body,
    out_shape=(
        jax.ShapeDtypeStruct((_N, _D), _F32),
        jax.ShapeDtypeStruct((_N, _D), _F32),
    ),
    compiler_params=pltpu.CompilerParams(vmem_limit_bytes=100 * 1024 * 1024),
)


def _post_norm(sp_ref, p_ref, h_ref, deg_ref, b_ref, g_ref, bt_ref):
    dinv = _dinv_from(deg_ref[...])
    sp = sp_ref[...]
    agg = dinv * (sp[0] + sp[1] + p_ref[...]) + b_ref[...][None, :]
    mu = jnp.mean(agg, axis=0)
    xc = agg - mu[None, :]
    var = jnp.mean(xc * xc, axis=0)
    y = xc * lax.rsqrt(var + _EPS)[None, :] * g_ref[...][None, :] + bt_ref[...][None, :]
    return jnp.maximum(y, 0.0) + h_ref[...], dinv


def _tc_norm_body(sp_ref, p_ref, h_ref, deg_ref, b_ref, g_ref, bt_ref, Wn_ref,
                  hn_ref, pn_ref):
    hn, dinv = _post_norm(sp_ref, p_ref, h_ref, deg_ref, b_ref, g_ref, bt_ref)
    hn_ref[...] = hn
    hw = jnp.dot(hn, Wn_ref[...], precision=_HI, preferred_element_type=_F32)
    pn_ref[...] = hw * dinv


_tc_norm = pl.pallas_call(
    _tc_norm_body,
    out_shape=(
        jax.ShapeDtypeStruct((_N, _D), _F32),
        jax.ShapeDtypeStruct((_N, _D), _F32),
    ),
    compiler_params=pltpu.CompilerParams(vmem_limit_bytes=100 * 1024 * 1024),
)


def _tc_final_body(sp_ref, p_ref, h_ref, deg_ref, b_ref, g_ref, bt_ref,
                   batch_ref, W1_ref, b1_ref, W2_ref, b2_ref, out_ref):
    hn, _ = _post_norm(sp_ref, p_ref, h_ref, deg_ref, b_ref, g_ref, bt_ref)
    seg = lax.broadcasted_iota(jnp.int32, (_G, _N), 0)
    onehot = (seg == batch_ref[...]).astype(_F32)
    sums = jnp.dot(onehot, hn, precision=_HI, preferred_element_type=_F32)
    counts = jnp.sum(onehot, axis=1)
    pooled = sums / jnp.maximum(counts, 1.0)[:, None]
    o = jnp.dot(pooled, W1_ref[...], precision=_HI, preferred_element_type=_F32)
    o = jnp.maximum(o + b1_ref[...][None, :], 0.0)
    o = jnp.dot(o, W2_ref[...], precision=_HI, preferred_element_type=_F32)
    out_ref[...] = o + b2_ref[...][None, :]


_tc_final = pl.pallas_call(
    _tc_final_body,
    out_shape=jax.ShapeDtypeStruct((_G, 1), _F32),
    compiler_params=pltpu.CompilerParams(vmem_limit_bytes=100 * 1024 * 1024),
)


def kernel(x, edge_index, batch, W_emb, b_emb, W_convs, b_convs, gammas, betas,
           W1, b1, W2, b2):
    srcw = edge_index[0].reshape(_NW, _EPW)
    dstw = edge_index[1].reshape(_NW, _EPW)
    nmain = _QE * _BE * _KE
    srcm = srcw[:, :nmain].reshape(_NW, _QE, _BE, _KE)
    srct = srcw[:, nmain:]
    dstm = dstw[:, :nmain].reshape(_NW, _QE, _BE, _KE)
    dstt = dstw[:, nmain:]
    degp = _deg_call(dstw.reshape(_NW, _SD, _KD))
    deg2 = degp[:, :, 0]
    h, p = _tc_emb(x, W_emb, b_emb, W_convs[0], deg2)
    out = None
    for i in range(_L):
        sp = _edge_call(p, srcm, srct, dstm, dstt)
        if i < _L - 1:
            h, p = _tc_norm(sp, p, h, deg2, b_convs[i], gammas[i], betas[i],
                            W_convs[i + 1])
        else:
            out = _tc_final(sp, p, h, deg2, b_convs[i], gammas[i], betas[i],
                            batch.reshape(1, _N), W1, b1, W2, b2)
    return out


# speedup vs baseline: 1.0559x; 1.0266x over previous
import jax
import jax.numpy as jnp
from jax import lax
from jax.experimental import pallas as pl
from jax.experimental.pallas import tpu as pltpu
from jax.experimental.pallas import tpu_sc as plsc

_N = 10000
_E = 320000
_D = 128
_L = 3
_G = 256
_EPS = 1e-5

_NC = 2
_NS = 16
_NW = _NC * _NS
_EPW = _E // _NW

_KE = 128
_SE = _EPW // _KE
_KT = _EPW - _SE * _KE

_KD = 80
_SD = _EPW // _KD
_BD = 4
_QD = _SD // _BD
_NPAD = 10240
_RPT = _NPAD // _NS
_DW = 16

_F32 = jnp.float32
_HI = lax.Precision.HIGHEST

_sc_mesh = plsc.VectorSubcoreMesh(
    core_axis_name="c", subcore_axis_name="s", num_cores=_NC, num_subcores=_NS
)


def _sc_deg_body(dstm_hbm, out_hbm, idx_v, zero_v, one_v, acc, sem):
    c = lax.axis_index("c")
    s = lax.axis_index("s")
    wid = c * _NS + s

    @pl.loop(0, _KD)
    def _fill(i):
        zero_v[i, :] = jnp.zeros((_DW,), _F32)
        one_v[i, :] = jnp.ones((_DW,), _F32)

    for t in range(_RPT // _KD):
        pltpu.sync_copy(zero_v, acc.at[pl.ds(s * _RPT + t * _KD, _KD)])
    plsc.subcore_barrier()

    pltpu.sync_copy(dstm_hbm.at[wid], idx_v)

    @pl.loop(0, _QD)
    def _scat(qq):
        j0 = _BD * qq
        ss = [pltpu.async_copy(one_v, acc.at[idx_v.at[j0 + q]], sem, add=True)
              for q in range(_BD)]
        for d in ss:
            d.wait()

    pltpu.sync_copy(one_v, acc.at[idx_v.at[_SD - 1]], add=True)

    plsc.subcore_barrier()
    pltpu.sync_copy(acc.at[pl.ds(s * _RPT, _RPT)], out_hbm.at[c, pl.ds(s * _RPT, _RPT)])


_deg_call = pl.kernel(
    _sc_deg_body,
    out_type=jax.ShapeDtypeStruct((_NC, _NPAD, _DW), _F32),
    mesh=_sc_mesh,
    scratch_types=[
        pltpu.VMEM((_SD, _KD), jnp.int32),
        pltpu.VMEM((_KD, _DW), _F32),
        pltpu.VMEM((_KD, _DW), _F32),
        pltpu.VMEM_SHARED((_NPAD, _DW), _F32),
        pltpu.SemaphoreType.DMA,
    ],
)


def _sc_edge_body(p_hbm, srcm_hbm, srct_hbm, dstm_hbm, dstt_hbm, out_hbm,
                  sidx, st_v, db0, db1, dt_v, buf0, buf1, acc,
                  gs0, gs1, ss0, ss1):
    c = lax.axis_index("c")
    s = lax.axis_index("s")
    wid = c * _NS + s

    @pl.loop(0, _KE)
    def _fill(i):
        for dd in range(_D // 16):
            buf0[i, pl.ds(dd * 16, 16)] = jnp.zeros((16,), _F32)

    for t in range(_RPT // _KE):
        pltpu.sync_copy(buf0, acc.at[pl.ds(s * _RPT + t * _KE, _KE)])
    plsc.subcore_barrier()

    pltpu.sync_copy(srcm_hbm.at[wid], sidx)
    pltpu.sync_copy(srct_hbm.at[wid], st_v)

    @pl.loop(0, _SE // 2)
    def _edge(jj):
        j0 = 2 * jj
        g0 = pltpu.async_copy(p_hbm.at[sidx.at[j0]], buf0, gs0)
        g1 = pltpu.async_copy(p_hbm.at[sidx.at[j0 + 1]], buf1, gs1)
        pltpu.sync_copy(dstm_hbm.at[wid, j0], db0)
        pltpu.sync_copy(dstm_hbm.at[wid, j0 + 1], db1)
        g0.wait()
        g1.wait()
        s0 = pltpu.async_copy(buf0, acc.at[db0], ss0, add=True)
        s1 = pltpu.async_copy(buf1, acc.at[db1], ss1, add=True)
        s0.wait()
        s1.wait()

    pltpu.sync_copy(dstt_hbm.at[wid], dt_v)
    pltpu.async_copy(p_hbm.at[st_v], buf0.at[pl.ds(0, _KT)], gs0).wait()
    pltpu.sync_copy(buf0.at[pl.ds(0, _KT)], acc.at[dt_v], add=True)

    plsc.subcore_barrier()
    pltpu.sync_copy(acc.at[pl.ds(s * _RPT, _RPT)], out_hbm.at[c, pl.ds(s * _RPT, _RPT)])


_edge_call = pl.kernel(
    _sc_edge_body,
    out_type=jax.ShapeDtypeStruct((_NC, _NPAD, _D), _F32),
    mesh=_sc_mesh,
    scratch_types=[
        pltpu.VMEM((_SE, _KE), jnp.int32),
        pltpu.VMEM((_KT,), jnp.int32),
        pltpu.VMEM((_KE,), jnp.int32),
        pltpu.VMEM((_KE,), jnp.int32),
        pltpu.VMEM((_KT,), jnp.int32),
        pltpu.VMEM((_KE, _D), _F32),
        pltpu.VMEM((_KE, _D), _F32),
        pltpu.VMEM_SHARED((_NPAD, _D), _F32),
        pltpu.SemaphoreType.DMA,
        pltpu.SemaphoreType.DMA,
        pltpu.SemaphoreType.DMA,
        pltpu.SemaphoreType.DMA,
    ],
)


def _dinv_from(deg2):
    deg = deg2[0, :_N] + deg2[1, :_N] + 1.0
    return (1.0 / jnp.sqrt(deg))[:, None]


def _tc_emb_body(x_ref, We_ref, be_ref, W0_ref, deg_ref, h_ref, p_ref):
    dinv = _dinv_from(deg_ref[...])
    h = jnp.dot(x_ref[...], We_ref[...], precision=_HI, preferred_element_type=_F32)
    h = h + be_ref[...][None, :]
    hw = jnp.dot(h, W0_ref[...], precision=_HI, preferred_element_type=_F32)
    h_ref[...] = h
    p_ref[...] = hw * dinv


_tc_emb = pl.pallas_call(
    _tc_emb_body,
    out_shape=(
        jax.ShapeDtypeStruct((_N, _D), _F32),
        jax.ShapeDtypeStruct((_N, _D), _F32),
    ),
    compiler_params=pltpu.CompilerParams(vmem_limit_bytes=100 * 1024 * 1024),
)


def _post_norm(sp_ref, p_ref, h_ref, deg_ref, b_ref, g_ref, bt_ref):
    dinv = _dinv_from(deg_ref[...])
    sp = sp_ref[...]
    agg = dinv * (sp[0, :_N, :] + sp[1, :_N, :] + p_ref[...]) + b_ref[...][None, :]
    mu = jnp.mean(agg, axis=0)
    xc = agg - mu[None, :]
    var = jnp.mean(xc * xc, axis=0)
    y = xc * lax.rsqrt(var + _EPS)[None, :] * g_ref[...][None, :] + bt_ref[...][None, :]
    return jnp.maximum(y, 0.0) + h_ref[...], dinv


def _tc_norm_body(sp_ref, p_ref, h_ref, deg_ref, b_ref, g_ref, bt_ref, Wn_ref,
                  hn_ref, pn_ref):
    hn, dinv = _post_norm(sp_ref, p_ref, h_ref, deg_ref, b_ref, g_ref, bt_ref)
    hn_ref[...] = hn
    hw = jnp.dot(hn, Wn_ref[...], precision=_HI, preferred_element_type=_F32)
    pn_ref[...] = hw * dinv


_tc_norm = pl.pallas_call(
    _tc_norm_body,
    out_shape=(
        jax.ShapeDtypeStruct((_N, _D), _F32),
        jax.ShapeDtypeStruct((_N, _D), _F32),
    ),
    compiler_params=pltpu.CompilerParams(vmem_limit_bytes=100 * 1024 * 1024),
)


def _tc_final_body(sp_ref, p_ref, h_ref, deg_ref, b_ref, g_ref, bt_ref,
                   batch_ref, W1_ref, b1_ref, W2_ref, b2_ref, out_ref):
    hn, _ = _post_norm(sp_ref, p_ref, h_ref, deg_ref, b_ref, g_ref, bt_ref)
    seg = lax.broadcasted_iota(jnp.int32, (_G, _N), 0)
    onehot = (seg == batch_ref[...]).astype(_F32)
    sums = jnp.dot(onehot, hn, precision=_HI, preferred_element_type=_F32)
    counts = jnp.sum(onehot, axis=1)
    pooled = sums / jnp.maximum(counts, 1.0)[:, None]
    o = jnp.dot(pooled, W1_ref[...], precision=_HI, preferred_element_type=_F32)
    o = jnp.maximum(o + b1_ref[...][None, :], 0.0)
    o = jnp.dot(o, W2_ref[...], precision=_HI, preferred_element_type=_F32)
    out_ref[...] = o + b2_ref[...][None, :]


_tc_final = pl.pallas_call(
    _tc_final_body,
    out_shape=jax.ShapeDtypeStruct((_G, 1), _F32),
    compiler_params=pltpu.CompilerParams(vmem_limit_bytes=100 * 1024 * 1024),
)


def kernel(x, edge_index, batch, W_emb, b_emb, W_convs, b_convs, gammas, betas,
           W1, b1, W2, b2):
    srcw = edge_index[0].reshape(_NW, _EPW)
    dstw = edge_index[1].reshape(_NW, _EPW)
    nmain = _SE * _KE
    srcm = srcw[:, :nmain].reshape(_NW, _SE, _KE)
    srct = srcw[:, nmain:]
    dstm = dstw[:, :nmain].reshape(_NW, _SE, _KE)
    dstt = dstw[:, nmain:]
    degp = _deg_call(dstw.reshape(_NW, _SD, _KD))
    deg2 = degp[:, :, 0]
    h, p = _tc_emb(x, W_emb, b_emb, W_convs[0], deg2)
    out = None
    for i in range(_L):
        sp = _edge_call(p, srcm, srct, dstm, dstt)
        if i < _L - 1:
            h, p = _tc_norm(sp, p, h, deg2, b_convs[i], gammas[i], betas[i],
                            W_convs[i + 1])
        else:
            out = _tc_final(sp, p, h, deg2, b_convs[i], gammas[i], betas[i],
                            batch.reshape(1, _N), W1, b1, W2, b2)
    return out
